# baseline (speedup 1.0000x reference)
import functools

import jax
import jax.numpy as jnp
from jax import lax
from jax.experimental import pallas as pl
from jax.experimental.pallas import tpu as pltpu
from jax.experimental.pallas import tpu_sc as plsc

_NODES = 26
_KPAD = 128
_B = 1024
_NC, _NS = 2, 16
_NW = _NC * _NS
_BPW = _B // _NW
_BT = 64
_RT = _BT * _NODES
_GRP = 16
_RG = _GRP * _NODES
_F32 = jnp.float32


def _sc_gather(table, indices, chunk):
    d = table.shape[1]
    nchunks = _BPW // chunk
    mesh = plsc.VectorSubcoreMesh(core_axis_name="c", subcore_axis_name="s")

    @functools.partial(
        pl.kernel,
        mesh=mesh,
        out_type=jax.ShapeDtypeStruct((_B, d), _F32),
        scratch_types=[
            pltpu.VMEM((chunk,), jnp.int32),
            pltpu.VMEM((chunk,), jnp.int32),
            pltpu.VMEM((chunk, d), _F32),
            pltpu.VMEM((chunk, d), _F32),
            pltpu.SemaphoreType.DMA,
            pltpu.SemaphoreType.DMA,
            pltpu.SemaphoreType.DMA,
            pltpu.SemaphoreType.DMA,
        ],
    )
    def gather_kernel(t_hbm, idx_hbm, out_hbm,
                      idx0, idx1, buf0, buf1, sg0, sg1, sw0, sw1):
        wid = lax.axis_index("s") * _NC + lax.axis_index("c")
        base = wid * _BPW
        idxs = (idx0, idx1)
        bufs = (buf0, buf1)
        gsems = (sg0, sg1)
        wsems = (sw0, sw1)

        def issue(c):
            b = c % 2
            pltpu.sync_copy(idx_hbm.at[pl.ds(base + c * chunk, chunk)],
                            idxs[b])
            return pltpu.async_copy(t_hbm.at[idxs[b]], bufs[b], gsems[b])

        gathers = [issue(0), issue(1)] + [None] * (nchunks - 2)
        writes = [None] * nchunks
        for c in range(nchunks):
            b = c % 2
            gathers[c].wait()
            writes[c] = pltpu.async_copy(
                bufs[b], out_hbm.at[pl.ds(base + c * chunk, chunk)],
                wsems[b])
            if c + 2 < nchunks:
                writes[c].wait()
                gathers[c + 2] = issue(c + 2)
        for c in (nchunks - 2, nchunks - 1):
            if writes[c] is not None:
                writes[c].wait()

    return gather_kernel(table, indices)


def _pad_body(k_ref, o_ref):
    blk = k_ref.shape[0]
    o_ref[:, :, :_NODES] = k_ref[...]
    o_ref[:, :, _NODES:] = jnp.zeros((blk, _NODES, _KPAD - _NODES), _F32)


def _tc_pad_k(k3d):
    n = k3d.shape[0]
    blk = 256
    return pl.pallas_call(
        _pad_body,
        grid=(n // blk,),
        in_specs=[pl.BlockSpec((blk, _NODES, _NODES), lambda i: (i, 0, 0))],
        out_specs=pl.BlockSpec((blk, _NODES, _KPAD), lambda i: (i, 0, 0)),
        out_shape=jax.ShapeDtypeStruct((n, _NODES, _KPAD), _F32),
    )(k3d)


def _tc_body(xb_ref, kb_ref, w0_ref, w1_ref, wl_ref, out_ref,
             s_ref, p_ref, m_ref):
    @pl.when(pl.program_id(0) == 0)
    def _init_masks():
        rowv = lax.broadcasted_iota(jnp.int32, (_BT, _RT), 0) * _NODES
        colv = lax.broadcasted_iota(jnp.int32, (_BT, _RT), 1)
        s_ref[...] = ((colv >= rowv) & (colv < rowv + _NODES)).astype(_F32)
        ir = lax.broadcasted_iota(jnp.int32, (_NODES, _NODES), 0)
        ic = lax.broadcasted_iota(jnp.int32, (_NODES, _NODES), 1)
        eye = (ir == ic).astype(_F32)
        for gj in range(_GRP):
            p_ref[:, gj * _NODES:(gj + 1) * _NODES] = eye
        rv = lax.broadcasted_iota(jnp.int32, (_GRP, _RG), 0) * _NODES
        cv = lax.broadcasted_iota(jnp.int32, (_GRP, _RG), 1)
        s16 = ((cv >= rv) & (cv < rv + _NODES)).astype(_F32)
        m_ref[...] = lax.dot_general(
            s16, s16, (((0,), (0,)), ((), ())),
            preferred_element_type=_F32)

    xb = xb_ref[...]
    kb = kb_ref[...][:, :_NODES]
    w0 = w0_ref[...]
    w1 = w1_ref[...]
    wl = wl_ref[...]
    s_mask = s_ref[...]
    p_tile = p_ref[...]
    m_mask = m_ref[...]

    dinv_r = lax.rsqrt(jnp.sum(kb, axis=1, keepdims=True))
    colsum = jnp.dot(s_mask, kb, preferred_element_type=_F32)
    dinv_b = lax.rsqrt(colsum)
    dinv_c = lax.dot_general(
        s_mask, dinv_b, (((0,), (0,)), ((), ())),
        preferred_element_type=_F32)
    a_rows = kb * dinv_r * dinv_c

    h0 = jnp.dot(xb, w0, preferred_element_type=_F32)

    h1_parts = []
    for g in range(_BT // _GRP):
        rows = slice(g * _RG, (g + 1) * _RG)
        bd = jnp.dot(a_rows[rows], p_tile,
                     preferred_element_type=_F32) * m_mask
        agg = lax.dot_general(
            bd, h0[rows], (((0,), (0,)), ((), ())),
            preferred_element_type=_F32)
        h1_parts.append(jnp.maximum(agg, 0.0))
    h1 = jnp.concatenate(h1_parts, axis=0)

    g1 = jnp.dot(h1, w1, preferred_element_type=_F32)
    wg1 = g1 * a_rows[:, 0:1]
    centers = jnp.maximum(
        jnp.dot(s_mask, wg1, preferred_element_type=_F32), 0.0)
    out_ref[...] = jnp.maximum(
        jnp.dot(centers, wl, preferred_element_type=_F32), 0.0)


def _tc_compute(xg2, kg2, W0, W1, Wlin):
    in_dim = xg2.shape[1]
    h0 = W0.shape[1]
    h1 = W1.shape[1]
    od = Wlin.shape[1]
    return pl.pallas_call(
        _tc_body,
        grid=(_B // _BT,),
        in_specs=[
            pl.BlockSpec((_RT, in_dim), lambda i: (i, 0)),
            pl.BlockSpec((_RT, _KPAD), lambda i: (i, 0)),
            pl.BlockSpec((in_dim, h0), lambda i: (0, 0)),
            pl.BlockSpec((h0, h1), lambda i: (0, 0)),
            pl.BlockSpec((h1, od), lambda i: (0, 0)),
        ],
        out_specs=pl.BlockSpec((_BT, od), lambda i: (i, 0)),
        out_shape=jax.ShapeDtypeStruct((_B, od), _F32),
        scratch_shapes=[
            pltpu.VMEM((_BT, _RT), _F32),
            pltpu.VMEM((_NODES, _RG), _F32),
            pltpu.VMEM((_RG, _RG), _F32),
        ],
    )(xg2, kg2, W0, W1, Wlin)


def kernel(indices, graph_x, kernel, W0, W1, Wlin):
    n, nodes, in_dim = graph_x.shape
    x2d = graph_x.reshape(n, nodes * in_dim)
    xg = _sc_gather(x2d, indices, 16)
    kp = _tc_pad_k(kernel)
    kg = _sc_gather(kp.reshape(n, nodes * _KPAD), indices, 16)
    xg2 = xg.reshape(_B * nodes, in_dim)
    kg2 = kg.reshape(_B * nodes, _KPAD)
    return _tc_compute(xg2, kg2, W0, W1, Wlin)


# speedup vs baseline: 2.6516x; 2.6516x over previous
import functools

import jax
import jax.numpy as jnp
from jax import lax
from jax.experimental import pallas as pl
from jax.experimental.pallas import tpu as pltpu
from jax.experimental.pallas import tpu_sc as plsc

_NODES = 26
_KPAD = 128
_B = 1024
_NC, _NS = 2, 16
_NW = _NC * _NS
_BPW = _B // _NW
_BT = 64
_RT = _BT * _NODES
_GRP = 16
_RG = _GRP * _NODES
_F32 = jnp.float32


def _sc_gather(table, indices, chunk):
    d = table.shape[1]
    nrows = indices.shape[0]
    per_w = nrows // _NW
    nchunks = per_w // chunk
    mesh = plsc.VectorSubcoreMesh(core_axis_name="c", subcore_axis_name="s")

    @functools.partial(
        pl.kernel,
        mesh=mesh,
        out_type=jax.ShapeDtypeStruct((nrows, d), _F32),
        scratch_types=[
            pltpu.VMEM((chunk,), jnp.int32),
            pltpu.VMEM((chunk,), jnp.int32),
            pltpu.VMEM((chunk, d), _F32),
            pltpu.VMEM((chunk, d), _F32),
            pltpu.SemaphoreType.DMA,
            pltpu.SemaphoreType.DMA,
            pltpu.SemaphoreType.DMA,
            pltpu.SemaphoreType.DMA,
        ],
    )
    def gather_kernel(t_hbm, idx_hbm, out_hbm,
                      idx0, idx1, buf0, buf1, sg0, sg1, sw0, sw1):
        wid = lax.axis_index("s") * _NC + lax.axis_index("c")
        base = wid * per_w
        idxs = (idx0, idx1)
        bufs = (buf0, buf1)
        gsems = (sg0, sg1)
        wsems = (sw0, sw1)

        def issue(c):
            b = c % 2
            pltpu.sync_copy(idx_hbm.at[pl.ds(base + c * chunk, chunk)],
                            idxs[b])
            return pltpu.async_copy(t_hbm.at[idxs[b]], bufs[b], gsems[b])

        gathers = [issue(0), issue(1)] + [None] * (nchunks - 2)
        writes = [None] * nchunks
        for c in range(nchunks):
            b = c % 2
            gathers[c].wait()
            writes[c] = pltpu.async_copy(
                bufs[b], out_hbm.at[pl.ds(base + c * chunk, chunk)],
                wsems[b])
            if c + 2 < nchunks:
                writes[c].wait()
                gathers[c + 2] = issue(c + 2)
        for c in (nchunks - 2, nchunks - 1):
            if writes[c] is not None:
                writes[c].wait()

    return gather_kernel(table, indices)


def _tc_body(xb_ref, kb_ref, w0_ref, w1_ref, wl_ref, out_ref,
             s_ref, p_ref, m_ref):
    @pl.when(pl.program_id(0) == 0)
    def _init_masks():
        rowv = lax.broadcasted_iota(jnp.int32, (_BT, _RT), 0) * _NODES
        colv = lax.broadcasted_iota(jnp.int32, (_BT, _RT), 1)
        s_ref[...] = ((colv >= rowv) & (colv < rowv + _NODES)).astype(_F32)
        ir = lax.broadcasted_iota(jnp.int32, (_NODES, _NODES), 0)
        ic = lax.broadcasted_iota(jnp.int32, (_NODES, _NODES), 1)
        eye = (ir == ic).astype(_F32)
        for gj in range(_GRP):
            p_ref[:, gj * _NODES:(gj + 1) * _NODES] = eye
        rv = lax.broadcasted_iota(jnp.int32, (_GRP, _RG), 0) * _NODES
        cv = lax.broadcasted_iota(jnp.int32, (_GRP, _RG), 1)
        s16 = ((cv >= rv) & (cv < rv + _NODES)).astype(_F32)
        m_ref[...] = lax.dot_general(
            s16, s16, (((0,), (0,)), ((), ())),
            preferred_element_type=_F32)

    xb = xb_ref[...]
    kb = kb_ref[...][:, :_NODES]
    w0 = w0_ref[...]
    w1 = w1_ref[...]
    wl = wl_ref[...]
    s_mask = s_ref[...]
    p_tile = p_ref[...]
    m_mask = m_ref[...]

    dinv_r = lax.rsqrt(jnp.sum(kb, axis=1, keepdims=True))
    colsum = jnp.dot(s_mask, kb, preferred_element_type=_F32)
    dinv_b = lax.rsqrt(colsum)
    dinv_c = lax.dot_general(
        s_mask, dinv_b, (((0,), (0,)), ((), ())),
        preferred_element_type=_F32)
    a_rows = kb * dinv_r * dinv_c

    h0 = jnp.dot(xb, w0, preferred_element_type=_F32)

    h1_parts = []
    for g in range(_BT // _GRP):
        rows = slice(g * _RG, (g + 1) * _RG)
        bd = jnp.dot(a_rows[rows], p_tile,
                     preferred_element_type=_F32) * m_mask
        agg = lax.dot_general(
            bd, h0[rows], (((0,), (0,)), ((), ())),
            preferred_element_type=_F32)
        h1_parts.append(jnp.maximum(agg, 0.0))
    h1 = jnp.concatenate(h1_parts, axis=0)

    g1 = jnp.dot(h1, w1, preferred_element_type=_F32)
    wg1 = g1 * a_rows[:, 0:1]
    centers = jnp.maximum(
        jnp.dot(s_mask, wg1, preferred_element_type=_F32), 0.0)
    out_ref[...] = jnp.maximum(
        jnp.dot(centers, wl, preferred_element_type=_F32), 0.0)


def _tc_compute(xg2, kg2, W0, W1, Wlin):
    in_dim = xg2.shape[1]
    h0 = W0.shape[1]
    h1 = W1.shape[1]
    od = Wlin.shape[1]
    return pl.pallas_call(
        _tc_body,
        grid=(_B // _BT,),
        in_specs=[
            pl.BlockSpec((_RT, in_dim), lambda i: (i, 0)),
            pl.BlockSpec((_RT, _KPAD), lambda i: (i, 0)),
            pl.BlockSpec((in_dim, h0), lambda i: (0, 0)),
            pl.BlockSpec((h0, h1), lambda i: (0, 0)),
            pl.BlockSpec((h1, od), lambda i: (0, 0)),
        ],
        out_specs=pl.BlockSpec((_BT, od), lambda i: (i, 0)),
        out_shape=jax.ShapeDtypeStruct((_B, od), _F32),
        scratch_shapes=[
            pltpu.VMEM((_BT, _RT), _F32),
            pltpu.VMEM((_NODES, _RG), _F32),
            pltpu.VMEM((_RG, _RG), _F32),
        ],
    )(xg2, kg2, W0, W1, Wlin)


def kernel(indices, graph_x, kernel, W0, W1, Wlin):
    n, nodes, in_dim = graph_x.shape
    xt = jnp.transpose(graph_x, (1, 0, 2)).reshape(nodes * n, in_dim)
    kp = jnp.pad(kernel, ((0, 0), (0, 0), (0, _KPAD - nodes)))
    kt = jnp.transpose(kp, (1, 0, 2)).reshape(nodes * n, _KPAD)
    idx_exp = (indices[:, None]
               + (jnp.arange(nodes, dtype=jnp.int32) * n)[None, :]
               ).reshape(-1)
    xg2 = _sc_gather(xt, idx_exp, 416)
    kg2 = _sc_gather(kt, idx_exp, 416)
    return _tc_compute(xg2, kg2, W0, W1, Wlin)
